# Initial kernel scaffold; baseline (speedup 1.0000x reference)
import jax
import jax.numpy as jnp
from jax.experimental import pallas as pl


def kernel(x, edge_index, edge_attr, edge_weight, eps, Wb, bb, W1, b1, W2, b2):
    raise NotImplementedError("write your pallas kernel here")


# speedup vs baseline: 3.0493x; 3.0493x over previous
import functools

import jax
import jax.numpy as jnp
from jax import lax
from jax.experimental import pallas as pl
from jax.experimental.pallas import tpu as pltpu
from jax.experimental.pallas import tpu_sc as plsc

N, E, D, DE = 10000, 320000, 128, 16

_info = plsc.get_sparse_core_info()
NC, NS = _info.num_cores, _info.num_subcores
NW = NC * NS
EPW = E // NW
K = 128
FULL = EPW // K
TAIL = EPW - FULL * K

_mesh = plsc.VectorSubcoreMesh(core_axis_name="c", subcore_axis_name="s")


@functools.partial(
    pl.kernel,
    mesh=_mesh,
    out_type=jax.ShapeDtypeStruct((E, D), jnp.float32),
    scratch_types=[
        pltpu.VMEM((K,), jnp.int32),
        pltpu.VMEM((K, D), jnp.float32),
        pltpu.VMEM((TAIL,), jnp.int32),
        pltpu.VMEM((TAIL, D), jnp.float32),
        pltpu.SemaphoreType.DMA,
    ],
)
def _gather_sc(x_hbm, src_hbm, out_hbm, idx_v, rows_v, idxt_v, rowst_v, sem):
    wid = lax.axis_index("s") * NC + lax.axis_index("c")
    base = wid * EPW

    def body(i, carry):
        off = base + i * K
        pltpu.sync_copy(src_hbm.at[pl.ds(off, K)], idx_v)
        pltpu.async_copy(x_hbm.at[idx_v], rows_v, sem).wait()
        pltpu.sync_copy(rows_v, out_hbm.at[pl.ds(off, K)])
        return carry

    lax.fori_loop(0, FULL, body, 0)
    off = base + FULL * K
    pltpu.sync_copy(src_hbm.at[pl.ds(off, TAIL)], idxt_v)
    pltpu.async_copy(x_hbm.at[idxt_v], rowst_v, sem).wait()
    pltpu.sync_copy(rowst_v, out_hbm.at[pl.ds(off, TAIL)])


@functools.partial(
    pl.kernel,
    mesh=_mesh,
    out_type=jax.ShapeDtypeStruct((NC * N, D), jnp.float32),
    scratch_types=[
        pltpu.VMEM((K,), jnp.int32),
        pltpu.VMEM((K, D), jnp.float32),
        pltpu.VMEM((TAIL,), jnp.int32),
        pltpu.VMEM((TAIL, D), jnp.float32),
        pltpu.VMEM_SHARED((N, D), jnp.float32),
        pltpu.SemaphoreType.DMA,
    ],
)
def _scatter_sc(msg_hbm, dst_hbm, zero_hbm, out_hbm,
                idx_v, rows_v, idxt_v, rowst_v, acc_sh, sem):
    c = lax.axis_index("c")
    s = lax.axis_index("s")
    wid = s * NC + c

    @pl.when(s == 0)
    def _init():
        pltpu.sync_copy(zero_hbm, acc_sh)

    plsc.subcore_barrier()
    base = wid * EPW

    def body(i, carry):
        off = base + i * K
        pltpu.sync_copy(dst_hbm.at[pl.ds(off, K)], idx_v)
        pltpu.sync_copy(msg_hbm.at[pl.ds(off, K)], rows_v)
        pltpu.sync_copy(rows_v, acc_sh.at[idx_v], add=True)
        return carry

    lax.fori_loop(0, FULL, body, 0)
    off = base + FULL * K
    pltpu.sync_copy(dst_hbm.at[pl.ds(off, TAIL)], idxt_v)
    pltpu.sync_copy(msg_hbm.at[pl.ds(off, TAIL)], rowst_v)
    pltpu.sync_copy(rowst_v, acc_sh.at[idxt_v], add=True)

    plsc.subcore_barrier()
    row0 = s * 632

    @pl.when(s < NS - 1)
    def _copy_main():
        pltpu.sync_copy(acc_sh.at[pl.ds(row0, 632)],
                        out_hbm.at[pl.ds(c * N + row0, 632)])

    @pl.when(s == NS - 1)
    def _copy_last():
        pltpu.sync_copy(acc_sh.at[pl.ds((NS - 1) * 632, N - (NS - 1) * 632)],
                        out_hbm.at[pl.ds(c * N + (NS - 1) * 632,
                                         N - (NS - 1) * 632)])


_SQRT_HALF = 0.7071067811865476


def _msg_body(attr_ref, gath_ref, w_ref, wb_ref, bb_ref, out_ref):
    emb = jnp.dot(attr_ref[...], wb_ref[...],
                  preferred_element_type=jnp.float32) + bb_ref[...]
    u = gath_ref[...] + emb
    g = 0.5 * u * (1.0 + lax.erf(u * _SQRT_HALF))
    out_ref[...] = g * w_ref[...]


def _fin_body(x_ref, p0_ref, p1_ref, epsb_ref, w1_ref, b1_ref, w2_ref,
              b2_ref, out_ref):
    h = x_ref[...] * epsb_ref[...] + p0_ref[0] + p1_ref[0]
    h1 = jnp.maximum(
        jnp.dot(h, w1_ref[...], preferred_element_type=jnp.float32)
        + b1_ref[...], 0.0)
    out_ref[...] = jnp.dot(h1, w2_ref[...],
                           preferred_element_type=jnp.float32) + b2_ref[...]


_BE = 2000
_BN = 2000


def _message_tc(gathered, edge_attr, edge_weight, Wb, bb):
    grid = (E // _BE,)
    return pl.pallas_call(
        _msg_body,
        grid=grid,
        in_specs=[
            pl.BlockSpec((_BE, DE), lambda i: (i, 0)),
            pl.BlockSpec((_BE, D), lambda i: (i, 0)),
            pl.BlockSpec((_BE, 1), lambda i: (i, 0)),
            pl.BlockSpec((DE, D), lambda i: (0, 0)),
            pl.BlockSpec((1, D), lambda i: (0, 0)),
        ],
        out_specs=pl.BlockSpec((_BE, D), lambda i: (i, 0)),
        out_shape=jax.ShapeDtypeStruct((E, D), jnp.float32),
    )(edge_attr, gathered, edge_weight, Wb, bb.reshape(1, D))


def _final_tc(x, parts, eps, W1, b1, W2, b2):
    grid = (N // _BN,)
    parts3 = parts.reshape(NC, N, D)
    epsb = jnp.broadcast_to((1.0 + eps).reshape(1, 1), (1, D))
    return pl.pallas_call(
        _fin_body,
        grid=grid,
        in_specs=[
            pl.BlockSpec((_BN, D), lambda i: (i, 0)),
            pl.BlockSpec((1, _BN, D), lambda i: (0, i, 0)),
            pl.BlockSpec((1, _BN, D), lambda i: (1, i, 0)),
            pl.BlockSpec((1, D), lambda i: (0, 0)),
            pl.BlockSpec((D, D), lambda i: (0, 0)),
            pl.BlockSpec((1, D), lambda i: (0, 0)),
            pl.BlockSpec((D, D), lambda i: (0, 0)),
            pl.BlockSpec((1, D), lambda i: (0, 0)),
        ],
        out_specs=pl.BlockSpec((_BN, D), lambda i: (i, 0)),
        out_shape=jax.ShapeDtypeStruct((N, D), jnp.float32),
    )(x, parts3, parts3, epsb, W1, b1.reshape(1, D), W2, b2.reshape(1, D))


def kernel(x, edge_index, edge_attr, edge_weight, eps, Wb, bb, W1, b1, W2, b2):
    src = edge_index[0].astype(jnp.int32)
    dst = edge_index[1].astype(jnp.int32)
    gathered = _gather_sc(x, src)
    msg = _message_tc(gathered, edge_attr, edge_weight, Wb, bb)
    zero = jnp.zeros((N, D), jnp.float32)
    parts = _scatter_sc(msg, dst, zero)
    return _final_tc(x, parts, eps, W1, b1, W2, b2)


# speedup vs baseline: 3.5970x; 1.1796x over previous
import functools

import jax
import jax.numpy as jnp
from jax import lax
from jax.experimental import pallas as pl
from jax.experimental.pallas import tpu as pltpu
from jax.experimental.pallas import tpu_sc as plsc

N, E, D, DE = 10000, 320000, 128, 16

_info = plsc.get_sparse_core_info()
NC, NS = _info.num_cores, _info.num_subcores
NW = NC * NS
EPW = E // NW
K = 80
CH = EPW // K
HALF = CH // 2

_mesh = plsc.VectorSubcoreMesh(core_axis_name="c", subcore_axis_name="s")


@functools.partial(
    pl.kernel,
    mesh=_mesh,
    out_type=jax.ShapeDtypeStruct((E, D), jnp.float32),
    scratch_types=[
        pltpu.VMEM((CH, K), jnp.int32),
        pltpu.VMEM((K, D), jnp.float32),
        pltpu.VMEM((K, D), jnp.float32),
        pltpu.VMEM_SHARED((N, D), jnp.float32),
        pltpu.SemaphoreType.DMA,
        pltpu.SemaphoreType.DMA,
    ],
    compiler_params=pltpu.CompilerParams(use_tc_tiling_on_sc=True),
)
def _gather_sc(x_hbm, src_hbm, out_hbm, idx_v, rows0_v, rows1_v, x_sh,
               sem0, sem1):
    c = lax.axis_index("c")
    s = lax.axis_index("s")
    wid = s * NC + c
    base = wid * EPW

    @pl.when(s == 0)
    def _stage_x():
        pltpu.sync_copy(x_hbm, x_sh)

    pltpu.sync_copy(src_hbm.at[wid], idx_v)
    plsc.subcore_barrier()

    def _gather(i, rows_v, sem):
        return pltpu.make_async_copy(x_sh.at[idx_v.at[i]], rows_v, sem)

    _gather(0, rows0_v, sem0).start()

    def body(j, carry):
        i0 = j * 2
        _gather(i0 + 1, rows1_v, sem1).start()
        _gather(i0, rows0_v, sem0).wait()
        pltpu.sync_copy(rows0_v, out_hbm.at[pl.ds(base + i0 * K, K)])
        _gather(i0 + 2, rows0_v, sem0).start()
        _gather(i0 + 1, rows1_v, sem1).wait()
        pltpu.sync_copy(rows1_v, out_hbm.at[pl.ds(base + (i0 + 1) * K, K)])
        return carry

    lax.fori_loop(0, HALF, body, 0)
    _gather(CH - 1, rows0_v, sem0).wait()
    pltpu.sync_copy(rows0_v, out_hbm.at[pl.ds(base + (CH - 1) * K, K)])


@functools.partial(
    pl.kernel,
    mesh=_mesh,
    out_type=jax.ShapeDtypeStruct((NC * N, D), jnp.float32),
    scratch_types=[
        pltpu.VMEM((CH, K), jnp.int32),
        pltpu.VMEM((K, D), jnp.float32),
        pltpu.VMEM((K, D), jnp.float32),
        pltpu.VMEM_SHARED((N, D), jnp.float32),
        pltpu.SemaphoreType.DMA,
        pltpu.SemaphoreType.DMA,
    ],
)
def _scatter_sc(msg_hbm, dst_hbm, zero_hbm, out_hbm,
                idx_v, rows0_v, rows1_v, acc_sh, sem0, sem1):
    c = lax.axis_index("c")
    s = lax.axis_index("s")
    wid = s * NC + c
    base = wid * EPW

    @pl.when(s == 0)
    def _init():
        pltpu.sync_copy(zero_hbm, acc_sh)

    pltpu.sync_copy(dst_hbm.at[wid], idx_v)
    plsc.subcore_barrier()

    def _load(i, rows_v, sem):
        return pltpu.make_async_copy(
            msg_hbm.at[pl.ds(base + i * K, K)], rows_v, sem)

    _load(0, rows0_v, sem0).start()

    def body(j, carry):
        i0 = j * 2
        _load(i0 + 1, rows1_v, sem1).start()
        _load(i0, rows0_v, sem0).wait()
        pltpu.sync_copy(rows0_v, acc_sh.at[idx_v.at[i0]], add=True)
        _load(i0 + 2, rows0_v, sem0).start()
        _load(i0 + 1, rows1_v, sem1).wait()
        pltpu.sync_copy(rows1_v, acc_sh.at[idx_v.at[i0 + 1]], add=True)
        return carry

    lax.fori_loop(0, HALF, body, 0)
    _load(CH - 1, rows0_v, sem0).wait()
    pltpu.sync_copy(rows0_v, acc_sh.at[idx_v.at[CH - 1]], add=True)

    plsc.subcore_barrier()
    row0 = s * 632

    @pl.when(s < NS - 1)
    def _copy_main():
        pltpu.sync_copy(acc_sh.at[pl.ds(row0, 632)],
                        out_hbm.at[pl.ds(c * N + row0, 632)])

    @pl.when(s == NS - 1)
    def _copy_last():
        pltpu.sync_copy(acc_sh.at[pl.ds((NS - 1) * 632, N - (NS - 1) * 632)],
                        out_hbm.at[pl.ds(c * N + (NS - 1) * 632,
                                         N - (NS - 1) * 632)])


_ERF_C = (7.97849541765013881e-01, -1.32836068516001965e-01,
          1.97701148098663425e-02, -2.27462177408516587e-03,
          1.99395291306342029e-04, -1.27959354062735235e-05,
          5.58364243155262004e-07, -1.45918875009488802e-08,
          1.70511978779626315e-10)


def _gelu_fast(u):
    r = jnp.clip(u, -4.0, 4.0)
    s = r * r
    p = jnp.float32(_ERF_C[8])
    for k in range(7, -1, -1):
        p = p * s + jnp.float32(_ERF_C[k])
    g = r * p
    return 0.5 * u * (1.0 + g)


def _msg_body(attr_ref, gath_ref, w_ref, wb_ref, bb_ref, out_ref):
    emb = jnp.dot(attr_ref[...], wb_ref[...],
                  preferred_element_type=jnp.float32) + bb_ref[...]
    u = gath_ref[...].astype(jnp.float32) + emb
    out_ref[...] = _gelu_fast(u) * w_ref[...]


def _fin_body(x_ref, p0_ref, p1_ref, epsb_ref, w1_ref, b1_ref, w2_ref,
              b2_ref, out_ref):
    h = x_ref[...] * epsb_ref[...] + p0_ref[0] + p1_ref[0]
    h1 = jnp.maximum(
        jnp.dot(h, w1_ref[...], preferred_element_type=jnp.float32)
        + b1_ref[...], 0.0)
    out_ref[...] = jnp.dot(h1, w2_ref[...],
                           preferred_element_type=jnp.float32) + b2_ref[...]


_BE = 2000
_BN = 2000


def _message_tc(gathered, edge_attr, edge_weight, Wb, bb):
    grid = (E // _BE,)
    return pl.pallas_call(
        _msg_body,
        grid=grid,
        in_specs=[
            pl.BlockSpec((_BE, DE), lambda i: (i, 0)),
            pl.BlockSpec((_BE, D), lambda i: (i, 0)),
            pl.BlockSpec((_BE, 1), lambda i: (i, 0)),
            pl.BlockSpec((DE, D), lambda i: (0, 0)),
            pl.BlockSpec((1, D), lambda i: (0, 0)),
        ],
        out_specs=pl.BlockSpec((_BE, D), lambda i: (i, 0)),
        out_shape=jax.ShapeDtypeStruct((E, D), jnp.float32),
    )(edge_attr, gathered, edge_weight, Wb, bb.reshape(1, D))


def _final_tc(x, parts, eps, W1, b1, W2, b2):
    grid = (N // _BN,)
    parts3 = parts.reshape(NC, N, D)
    epsb = jnp.broadcast_to((1.0 + eps).reshape(1, 1), (1, D))
    return pl.pallas_call(
        _fin_body,
        grid=grid,
        in_specs=[
            pl.BlockSpec((_BN, D), lambda i: (i, 0)),
            pl.BlockSpec((1, _BN, D), lambda i: (0, i, 0)),
            pl.BlockSpec((1, _BN, D), lambda i: (1, i, 0)),
            pl.BlockSpec((1, D), lambda i: (0, 0)),
            pl.BlockSpec((D, D), lambda i: (0, 0)),
            pl.BlockSpec((1, D), lambda i: (0, 0)),
            pl.BlockSpec((D, D), lambda i: (0, 0)),
            pl.BlockSpec((1, D), lambda i: (0, 0)),
        ],
        out_specs=pl.BlockSpec((_BN, D), lambda i: (i, 0)),
        out_shape=jax.ShapeDtypeStruct((N, D), jnp.float32),
    )(x, parts3, parts3, epsb, W1, b1.reshape(1, D), W2, b2.reshape(1, D))


def kernel(x, edge_index, edge_attr, edge_weight, eps, Wb, bb, W1, b1, W2, b2):
    src = edge_index[0].astype(jnp.int32).reshape(NW, CH, K)
    dst = edge_index[1].astype(jnp.int32).reshape(NW, CH, K)
    gathered = _gather_sc(x, src)
    msg = _message_tc(gathered, edge_attr, edge_weight, Wb, bb)
    zero = jnp.zeros((N, D), jnp.float32)
    parts = _scatter_sc(msg, dst, zero)
    return _final_tc(x, parts, eps, W1, b1, W2, b2)


# speedup vs baseline: 3.7322x; 1.0376x over previous
import functools

import jax
import jax.numpy as jnp
from jax import lax
from jax.experimental import pallas as pl
from jax.experimental.pallas import tpu as pltpu
from jax.experimental.pallas import tpu_sc as plsc

N, E, D, DE = 10000, 320000, 128, 16

_info = plsc.get_sparse_core_info()
NC, NS = _info.num_cores, _info.num_subcores
NW = NC * NS
K = 80
S = 5
ES = E // S
RPW = ES // (NW * K)
RH = RPW // 2

_mesh = plsc.VectorSubcoreMesh(core_axis_name="c", subcore_axis_name="s")

_ROWS_MAIN = 632
_ROWS_LAST = N - (NS - 1) * _ROWS_MAIN


def _stage_rows(s, src_ref, dst_ref, src_off=0, dst_off=0):
    @pl.when(s < NS - 1)
    def _main():
        pltpu.sync_copy(
            src_ref.at[pl.ds(src_off + s * _ROWS_MAIN, _ROWS_MAIN)],
            dst_ref.at[pl.ds(dst_off + s * _ROWS_MAIN, _ROWS_MAIN)])

    @pl.when(s == NS - 1)
    def _last():
        pltpu.sync_copy(
            src_ref.at[pl.ds(src_off + (NS - 1) * _ROWS_MAIN, _ROWS_LAST)],
            dst_ref.at[pl.ds(dst_off + (NS - 1) * _ROWS_MAIN, _ROWS_LAST)])


@functools.partial(
    pl.kernel,
    mesh=_mesh,
    out_type=jax.ShapeDtypeStruct((ES, D), jnp.float32),
    scratch_types=[
        pltpu.VMEM((RPW, K), jnp.int32),
        pltpu.VMEM((K, D), jnp.float32),
        pltpu.VMEM((K, D), jnp.float32),
        pltpu.VMEM_SHARED((N, D), jnp.float32),
        pltpu.SemaphoreType.DMA,
        pltpu.SemaphoreType.DMA,
    ],
)
def _gather_sc(x_hbm, src_hbm, out_hbm, idx_v, rows0_v, rows1_v, x_sh,
               sem0, sem1):
    c = lax.axis_index("c")
    s = lax.axis_index("s")
    wid = s * NC + c

    _stage_rows(s, x_hbm, x_sh)
    pltpu.sync_copy(src_hbm.at[wid], idx_v)
    plsc.subcore_barrier()

    def _gather(i, rows_v, sem):
        return pltpu.make_async_copy(x_sh.at[idx_v.at[i]], rows_v, sem)

    def _out(i, rows_v):
        pltpu.sync_copy(rows_v, out_hbm.at[pl.ds((wid * RPW + i) * K, K)])

    _gather(0, rows0_v, sem0).start()

    def body(j, carry):
        i0 = j * 2
        _gather(i0 + 1, rows1_v, sem1).start()
        _gather(i0, rows0_v, sem0).wait()
        _out(i0, rows0_v)
        _gather(i0 + 2, rows0_v, sem0).start()
        _gather(i0 + 1, rows1_v, sem1).wait()
        _out(i0 + 1, rows1_v)
        return carry

    lax.fori_loop(0, RH, body, 0)
    _gather(RPW - 1, rows0_v, sem0).wait()
    _out(RPW - 1, rows0_v)


def _scatter_chunks(wid, msg_refs, dst_refs, idx_v, rows0_v, rows1_v,
                    acc_sh, sem0, sem1):
    for msg_hbm, dst_hbm in zip(msg_refs, dst_refs):
        pltpu.sync_copy(dst_hbm.at[wid], idx_v)

        def _load(i, rows_v, sem, _m=msg_hbm):
            return pltpu.make_async_copy(
                _m.at[pl.ds((wid * RPW + i) * K, K)], rows_v, sem)

        def _add(i, rows_v):
            pltpu.sync_copy(rows_v, acc_sh.at[idx_v.at[i]], add=True)

        _load(0, rows0_v, sem0).start()

        def body(j, carry, _load=_load, _add=_add):
            i0 = j * 2
            _load(i0 + 1, rows1_v, sem1).start()
            _load(i0, rows0_v, sem0).wait()
            _add(i0, rows0_v)
            _load(i0 + 2, rows0_v, sem0).start()
            _load(i0 + 1, rows1_v, sem1).wait()
            _add(i0 + 1, rows1_v)
            return carry

        lax.fori_loop(0, RH, body, 0)
        _load(RPW - 1, rows0_v, sem0).wait()
        _add(RPW - 1, rows0_v)


_SC_SCRATCH = [
    pltpu.VMEM((RPW, K), jnp.int32),
    pltpu.VMEM((K, D), jnp.float32),
    pltpu.VMEM((K, D), jnp.float32),
    pltpu.VMEM_SHARED((N, D), jnp.float32),
    pltpu.SemaphoreType.DMA,
    pltpu.SemaphoreType.DMA,
]
_PART_T = jax.ShapeDtypeStruct((NC * N, D), jnp.float32)


@functools.partial(pl.kernel, mesh=_mesh, out_type=_PART_T,
                   scratch_types=_SC_SCRATCH)
def _scatter3_sc(m0_hbm, m1_hbm, m2_hbm, d0_hbm, d1_hbm, d2_hbm, zero_hbm,
                 out_hbm, idx_v, rows0_v, rows1_v, acc_sh, sem0, sem1):
    c = lax.axis_index("c")
    s = lax.axis_index("s")
    wid = s * NC + c
    _stage_rows(s, zero_hbm, acc_sh)
    plsc.subcore_barrier()
    _scatter_chunks(wid, (m0_hbm, m1_hbm, m2_hbm), (d0_hbm, d1_hbm, d2_hbm),
                    idx_v, rows0_v, rows1_v, acc_sh, sem0, sem1)
    plsc.subcore_barrier()
    _stage_rows(s, acc_sh, out_hbm, dst_off=c * N)


@functools.partial(pl.kernel, mesh=_mesh, out_type=_PART_T,
                   scratch_types=_SC_SCRATCH)
def _scatter2_sc(m3_hbm, m4_hbm, d3_hbm, d4_hbm, prev_hbm,
                 out_hbm, idx_v, rows0_v, rows1_v, acc_sh, sem0, sem1):
    c = lax.axis_index("c")
    s = lax.axis_index("s")
    wid = s * NC + c
    _stage_rows(s, prev_hbm, acc_sh, src_off=c * N)
    plsc.subcore_barrier()
    _scatter_chunks(wid, (m3_hbm, m4_hbm), (d3_hbm, d4_hbm),
                    idx_v, rows0_v, rows1_v, acc_sh, sem0, sem1)
    plsc.subcore_barrier()
    _stage_rows(s, acc_sh, out_hbm, dst_off=c * N)


_SQRT_HALF = 0.7071067811865476


def _msg_body(attr_ref, gath_ref, w_ref, wb_ref, bb_ref, out_ref):
    emb = jnp.dot(attr_ref[...], wb_ref[...],
                  preferred_element_type=jnp.float32) + bb_ref[...]
    u = gath_ref[0] + emb
    g = 0.5 * u * (1.0 + lax.erf(u * _SQRT_HALF))
    out_ref[...] = g * w_ref[...]


def _fin_body(x_ref, p0_ref, p1_ref, epsb_ref, w1_ref, b1_ref, w2_ref,
              b2_ref, out_ref):
    h = x_ref[...] * epsb_ref[...] + p0_ref[0] + p1_ref[0]
    h1 = jnp.maximum(
        jnp.dot(h, w1_ref[...], preferred_element_type=jnp.float32)
        + b1_ref[...], 0.0)
    out_ref[...] = jnp.dot(h1, w2_ref[...],
                           preferred_element_type=jnp.float32) + b2_ref[...]


_BE = 4000
_BN = 2000


def _message_tc(gathered, edge_attr, edge_weight, Wb, bb2):
    grid = (ES // _BE,)
    return pl.pallas_call(
        _msg_body,
        grid=grid,
        in_specs=[
            pl.BlockSpec((_BE, DE), lambda i: (i, 0)),
            pl.BlockSpec((1, _BE, D), lambda i: (0, i, 0)),
            pl.BlockSpec((_BE, 1), lambda i: (i, 0)),
            pl.BlockSpec((DE, D), lambda i: (0, 0)),
            pl.BlockSpec((1, D), lambda i: (0, 0)),
        ],
        out_specs=pl.BlockSpec((_BE, D), lambda i: (i, 0)),
        out_shape=jax.ShapeDtypeStruct((ES, D), jnp.float32),
    )(edge_attr, gathered.reshape(1, ES, D), edge_weight, Wb, bb2)


def _final_tc(x, parts, eps, W1, b1, W2, b2):
    grid = (N // _BN,)
    parts3 = parts.reshape(NC, N, D)
    epsb = jnp.broadcast_to((1.0 + eps).reshape(1, 1), (1, D))
    return pl.pallas_call(
        _fin_body,
        grid=grid,
        in_specs=[
            pl.BlockSpec((_BN, D), lambda i: (i, 0)),
            pl.BlockSpec((1, _BN, D), lambda i: (0, i, 0)),
            pl.BlockSpec((1, _BN, D), lambda i: (1, i, 0)),
            pl.BlockSpec((1, D), lambda i: (0, 0)),
            pl.BlockSpec((D, D), lambda i: (0, 0)),
            pl.BlockSpec((1, D), lambda i: (0, 0)),
            pl.BlockSpec((D, D), lambda i: (0, 0)),
            pl.BlockSpec((1, D), lambda i: (0, 0)),
        ],
        out_specs=pl.BlockSpec((_BN, D), lambda i: (i, 0)),
        out_shape=jax.ShapeDtypeStruct((N, D), jnp.float32),
    )(x, parts3, parts3, epsb, W1, b1.reshape(1, D), W2, b2.reshape(1, D))


def kernel(x, edge_index, edge_attr, edge_weight, eps, Wb, bb, W1, b1, W2, b2):
    src4 = edge_index[0].astype(jnp.int32).reshape(S, NW, RPW, K)
    dst4 = edge_index[1].astype(jnp.int32).reshape(S, NW, RPW, K)
    attr5 = edge_attr.reshape(S, ES, DE)
    w5 = edge_weight.reshape(S, ES, 1)
    bb2 = bb.reshape(1, D)
    msgs = []
    for k in range(S):
        g_k = _gather_sc(x, src4[k])
        msgs.append(_message_tc(g_k, attr5[k], w5[k], Wb, bb2))
    zero = jnp.zeros((N, D), jnp.float32)
    parts_a = _scatter3_sc(msgs[0], msgs[1], msgs[2],
                           dst4[0], dst4[1], dst4[2], zero)
    parts = _scatter2_sc(msgs[3], msgs[4], dst4[3], dst4[4], parts_a)
    return _final_tc(x, parts, eps, W1, b1, W2, b2)


# speedup vs baseline: 4.0422x; 1.0831x over previous
import functools

import jax
import jax.numpy as jnp
from jax import lax
from jax.experimental import pallas as pl
from jax.experimental.pallas import tpu as pltpu
from jax.experimental.pallas import tpu_sc as plsc

N, E, D, DE = 10000, 320000, 128, 16

_info = plsc.get_sparse_core_info()
NC, NS = _info.num_cores, _info.num_subcores
NW = NC * NS
K = 80
S = 5
ES = E // S
RPW = ES // (NW * K)
RH = RPW // 2

_mesh = plsc.VectorSubcoreMesh(core_axis_name="c", subcore_axis_name="s")

_ROWS_MAIN = 632
_ROWS_LAST = N - (NS - 1) * _ROWS_MAIN


def _stage_rows(s, src_ref, dst_ref, src_off=0, dst_off=0):
    @pl.when(s < NS - 1)
    def _main():
        pltpu.sync_copy(
            src_ref.at[pl.ds(src_off + s * _ROWS_MAIN, _ROWS_MAIN)],
            dst_ref.at[pl.ds(dst_off + s * _ROWS_MAIN, _ROWS_MAIN)])

    @pl.when(s == NS - 1)
    def _last():
        pltpu.sync_copy(
            src_ref.at[pl.ds(src_off + (NS - 1) * _ROWS_MAIN, _ROWS_LAST)],
            dst_ref.at[pl.ds(dst_off + (NS - 1) * _ROWS_MAIN, _ROWS_LAST)])


@functools.partial(
    pl.kernel,
    mesh=_mesh,
    out_type=jax.ShapeDtypeStruct((ES, D), jnp.float32),
    scratch_types=[
        pltpu.VMEM((RPW, K), jnp.int32),
        pltpu.VMEM((K, D), jnp.float32),
        pltpu.VMEM((K, D), jnp.float32),
        pltpu.SemaphoreType.DMA,
        pltpu.SemaphoreType.DMA,
    ],
)
def _gather_sc(x_hbm, src_hbm, out_hbm, idx_v, rows0_v, rows1_v,
               sem0, sem1):
    c = lax.axis_index("c")
    s = lax.axis_index("s")
    wid = s * NC + c

    pltpu.sync_copy(src_hbm.at[wid], idx_v)

    def _gather(i, rows_v, sem):
        return pltpu.make_async_copy(x_hbm.at[idx_v.at[i]], rows_v, sem)

    def _out(i, rows_v):
        pltpu.sync_copy(rows_v, out_hbm.at[pl.ds((wid * RPW + i) * K, K)])

    _gather(0, rows0_v, sem0).start()

    def body(j, carry):
        i0 = j * 2
        _gather(i0 + 1, rows1_v, sem1).start()
        _gather(i0, rows0_v, sem0).wait()
        _out(i0, rows0_v)
        _gather(i0 + 2, rows0_v, sem0).start()
        _gather(i0 + 1, rows1_v, sem1).wait()
        _out(i0 + 1, rows1_v)
        return carry

    lax.fori_loop(0, RH, body, 0)
    _gather(RPW - 1, rows0_v, sem0).wait()
    _out(RPW - 1, rows0_v)


def _scatter_chunks(wid, msg_refs, dst_refs, idx_v, rows0_v, rows1_v,
                    acc_sh, sem0, sem1):
    for msg_hbm, dst_hbm in zip(msg_refs, dst_refs):
        pltpu.sync_copy(dst_hbm.at[wid], idx_v)

        def _load(i, rows_v, sem, _m=msg_hbm):
            return pltpu.make_async_copy(
                _m.at[pl.ds((wid * RPW + i) * K, K)], rows_v, sem)

        def _add(i, rows_v):
            pltpu.sync_copy(rows_v, acc_sh.at[idx_v.at[i]], add=True)

        _load(0, rows0_v, sem0).start()

        def body(j, carry, _load=_load, _add=_add):
            i0 = j * 2
            _load(i0 + 1, rows1_v, sem1).start()
            _load(i0, rows0_v, sem0).wait()
            _add(i0, rows0_v)
            _load(i0 + 2, rows0_v, sem0).start()
            _load(i0 + 1, rows1_v, sem1).wait()
            _add(i0 + 1, rows1_v)
            return carry

        lax.fori_loop(0, RH, body, 0)
        _load(RPW - 1, rows0_v, sem0).wait()
        _add(RPW - 1, rows0_v)


_SC_SCRATCH = [
    pltpu.VMEM((RPW, K), jnp.int32),
    pltpu.VMEM((K, D), jnp.float32),
    pltpu.VMEM((K, D), jnp.float32),
    pltpu.VMEM_SHARED((N, D), jnp.float32),
    pltpu.SemaphoreType.DMA,
    pltpu.SemaphoreType.DMA,
]
_PART_T = jax.ShapeDtypeStruct((NC * N, D), jnp.float32)


@functools.partial(pl.kernel, mesh=_mesh, out_type=_PART_T,
                   scratch_types=_SC_SCRATCH)
def _scatter3_sc(m0_hbm, m1_hbm, m2_hbm, d0_hbm, d1_hbm, d2_hbm, zero_hbm,
                 out_hbm, idx_v, rows0_v, rows1_v, acc_sh, sem0, sem1):
    c = lax.axis_index("c")
    s = lax.axis_index("s")
    wid = s * NC + c
    _stage_rows(s, zero_hbm, acc_sh)
    plsc.subcore_barrier()
    _scatter_chunks(wid, (m0_hbm, m1_hbm, m2_hbm), (d0_hbm, d1_hbm, d2_hbm),
                    idx_v, rows0_v, rows1_v, acc_sh, sem0, sem1)
    plsc.subcore_barrier()
    _stage_rows(s, acc_sh, out_hbm, dst_off=c * N)


@functools.partial(pl.kernel, mesh=_mesh, out_type=_PART_T,
                   scratch_types=_SC_SCRATCH)
def _scatter2_sc(m3_hbm, m4_hbm, d3_hbm, d4_hbm, prev_hbm,
                 out_hbm, idx_v, rows0_v, rows1_v, acc_sh, sem0, sem1):
    c = lax.axis_index("c")
    s = lax.axis_index("s")
    wid = s * NC + c
    _stage_rows(s, prev_hbm, acc_sh, src_off=c * N)
    plsc.subcore_barrier()
    _scatter_chunks(wid, (m3_hbm, m4_hbm), (d3_hbm, d4_hbm),
                    idx_v, rows0_v, rows1_v, acc_sh, sem0, sem1)
    plsc.subcore_barrier()
    _stage_rows(s, acc_sh, out_hbm, dst_off=c * N)


_SQRT_HALF = 0.7071067811865476


def _msg_body(attr_ref, gath_ref, w_ref, wb_ref, bb_ref, out_ref):
    emb = jnp.dot(attr_ref[...], wb_ref[...],
                  preferred_element_type=jnp.float32) + bb_ref[...]
    u = gath_ref[...] + emb
    g = 0.5 * u * (1.0 + lax.erf(u * _SQRT_HALF))
    w = w_ref[pl.ds(pl.program_id(0) * _BE, _BE)]
    out_ref[...] = g * w[:, None]


def _fin_body(x_ref, p0_ref, p1_ref, epsb_ref, w1_ref, b1_ref, w2_ref,
              b2_ref, out_ref):
    h = x_ref[...] * epsb_ref[...] + p0_ref[0] + p1_ref[0]
    h1 = jnp.maximum(
        jnp.dot(h, w1_ref[...], preferred_element_type=jnp.float32)
        + b1_ref[...], 0.0)
    out_ref[...] = jnp.dot(h1, w2_ref[...],
                           preferred_element_type=jnp.float32) + b2_ref[...]


_BE = 6400
_BN = 2000


def _message_tc(gathered, edge_attr, edge_weight, Wb, bb2):
    grid = (ES // _BE,)
    return pl.pallas_call(
        _msg_body,
        grid=grid,
        in_specs=[
            pl.BlockSpec((_BE, DE), lambda i: (i, 0)),
            pl.BlockSpec((_BE, D), lambda i: (i, 0)),
            pl.BlockSpec((ES,), lambda i: (0,)),
            pl.BlockSpec((DE, D), lambda i: (0, 0)),
            pl.BlockSpec((1, D), lambda i: (0, 0)),
        ],
        out_specs=pl.BlockSpec((_BE, D), lambda i: (i, 0)),
        out_shape=jax.ShapeDtypeStruct((ES, D), jnp.float32),
    )(edge_attr, gathered, edge_weight, Wb, bb2)


def _final_tc(x, parts, eps, W1, b1, W2, b2):
    grid = (N // _BN,)
    parts3 = parts.reshape(NC, N, D)
    epsb = jnp.broadcast_to((1.0 + eps).reshape(1, 1), (1, D))
    return pl.pallas_call(
        _fin_body,
        grid=grid,
        in_specs=[
            pl.BlockSpec((_BN, D), lambda i: (i, 0)),
            pl.BlockSpec((1, _BN, D), lambda i: (0, i, 0)),
            pl.BlockSpec((1, _BN, D), lambda i: (1, i, 0)),
            pl.BlockSpec((1, D), lambda i: (0, 0)),
            pl.BlockSpec((D, D), lambda i: (0, 0)),
            pl.BlockSpec((1, D), lambda i: (0, 0)),
            pl.BlockSpec((D, D), lambda i: (0, 0)),
            pl.BlockSpec((1, D), lambda i: (0, 0)),
        ],
        out_specs=pl.BlockSpec((_BN, D), lambda i: (i, 0)),
        out_shape=jax.ShapeDtypeStruct((N, D), jnp.float32),
    )(x, parts3, parts3, epsb, W1, b1.reshape(1, D), W2, b2.reshape(1, D))


def kernel(x, edge_index, edge_attr, edge_weight, eps, Wb, bb, W1, b1, W2, b2):
    src4 = edge_index[0].astype(jnp.int32).reshape(S, NW, RPW, K)
    dst4 = edge_index[1].astype(jnp.int32).reshape(S, NW, RPW, K)
    attr5 = edge_attr.reshape(S, ES, DE)
    w5 = edge_weight.reshape(S, ES)
    bb2 = bb.reshape(1, D)
    msgs = []
    for k in range(S):
        g_k = _gather_sc(x, src4[k])
        msgs.append(_message_tc(g_k, attr5[k], w5[k], Wb, bb2))
    zero = jnp.zeros((N, D), jnp.float32)
    parts_a = _scatter3_sc(msgs[0], msgs[1], msgs[2],
                           dst4[0], dst4[1], dst4[2], zero)
    parts = _scatter2_sc(msgs[3], msgs[4], dst4[3], dst4[4], parts_a)
    return _final_tc(x, parts, eps, W1, b1, W2, b2)


# speedup vs baseline: 5.9937x; 1.4828x over previous
import functools

import jax
import jax.numpy as jnp
from jax import lax
from jax.experimental import pallas as pl
from jax.experimental.pallas import tpu as pltpu
from jax.experimental.pallas import tpu_sc as plsc

N, E, D, DE = 10000, 320000, 128, 16

_info = plsc.get_sparse_core_info()
NC, NS = _info.num_cores, _info.num_subcores
NW = NC * NS
K = 80
S = 5
ES = E // S
RPW = ES // (NW * K)
RH = RPW // 2

_mesh = plsc.VectorSubcoreMesh(core_axis_name="c", subcore_axis_name="s")

_ROWS_MAIN = 632
_ROWS_LAST = N - (NS - 1) * _ROWS_MAIN


def _stage_rows(s, src_ref, dst_ref, src_off=0, dst_off=0):
    @pl.when(s < NS - 1)
    def _main():
        pltpu.sync_copy(
            src_ref.at[pl.ds(src_off + s * _ROWS_MAIN, _ROWS_MAIN)],
            dst_ref.at[pl.ds(dst_off + s * _ROWS_MAIN, _ROWS_MAIN)])

    @pl.when(s == NS - 1)
    def _last():
        pltpu.sync_copy(
            src_ref.at[pl.ds(src_off + (NS - 1) * _ROWS_MAIN, _ROWS_LAST)],
            dst_ref.at[pl.ds(dst_off + (NS - 1) * _ROWS_MAIN, _ROWS_LAST)])


@functools.partial(
    pl.kernel,
    mesh=_mesh,
    out_type=jax.ShapeDtypeStruct((ES, D), jnp.float32),
    scratch_types=[
        pltpu.VMEM((RPW, K), jnp.int32),
        pltpu.VMEM((K, D), jnp.float32),
        pltpu.VMEM((K, D), jnp.float32),
        pltpu.VMEM_SHARED((N, D), jnp.float32),
        pltpu.SemaphoreType.DMA,
        pltpu.SemaphoreType.DMA,
    ],
)
def _gather_sc(x_hbm, src_hbm, out_hbm, idx_v, rows0_v, rows1_v, x_sh,
               sem0, sem1):
    c = lax.axis_index("c")
    s = lax.axis_index("s")
    wid = s * NC + c

    _stage_rows(s, x_hbm, x_sh)
    pltpu.sync_copy(src_hbm.at[wid], idx_v)
    plsc.subcore_barrier()

    def _gather(i, rows_v, sem):
        return pltpu.make_async_copy(x_sh.at[idx_v.at[i]], rows_v, sem)

    def _out(i, rows_v):
        pltpu.sync_copy(rows_v, out_hbm.at[pl.ds((wid * RPW + i) * K, K)])

    _gather(0, rows0_v, sem0).start()

    def body(j, carry):
        i0 = j * 2
        _gather(i0 + 1, rows1_v, sem1).start()
        _gather(i0, rows0_v, sem0).wait()
        _out(i0, rows0_v)
        _gather(i0 + 2, rows0_v, sem0).start()
        _gather(i0 + 1, rows1_v, sem1).wait()
        _out(i0 + 1, rows1_v)
        return carry

    lax.fori_loop(0, RH, body, 0)
    _gather(RPW - 1, rows0_v, sem0).wait()
    _out(RPW - 1, rows0_v)


def _scatter_chunks(wid, msg_refs, dst_refs, idx_v, rows0_v, rows1_v,
                    acc_sh, sem0, sem1):
    for msg_hbm, dst_hbm in zip(msg_refs, dst_refs):
        pltpu.sync_copy(dst_hbm.at[wid], idx_v)

        def _load(i, rows_v, sem, _m=msg_hbm):
            return pltpu.make_async_copy(
                _m.at[pl.ds((wid * RPW + i) * K, K)], rows_v, sem)

        def _add(i, rows_v):
            pltpu.sync_copy(rows_v, acc_sh.at[idx_v.at[i]], add=True)

        _load(0, rows0_v, sem0).start()

        def body(j, carry, _load=_load, _add=_add):
            i0 = j * 2
            _load(i0 + 1, rows1_v, sem1).start()
            _load(i0, rows0_v, sem0).wait()
            _add(i0, rows0_v)
            _load(i0 + 2, rows0_v, sem0).start()
            _load(i0 + 1, rows1_v, sem1).wait()
            _add(i0 + 1, rows1_v)
            return carry

        lax.fori_loop(0, RH, body, 0)
        _load(RPW - 1, rows0_v, sem0).wait()
        _add(RPW - 1, rows0_v)


_SC_SCRATCH = [
    pltpu.VMEM((RPW, K), jnp.int32),
    pltpu.VMEM((K, D), jnp.float32),
    pltpu.VMEM((K, D), jnp.float32),
    pltpu.VMEM_SHARED((N, D), jnp.float32),
    pltpu.SemaphoreType.DMA,
    pltpu.SemaphoreType.DMA,
]
_PART_T = jax.ShapeDtypeStruct((NC * N, D), jnp.float32)


@functools.partial(pl.kernel, mesh=_mesh, out_type=_PART_T,
                   scratch_types=_SC_SCRATCH)
def _scatter3_sc(m0_hbm, m1_hbm, m2_hbm, d0_hbm, d1_hbm, d2_hbm, zero_hbm,
                 out_hbm, idx_v, rows0_v, rows1_v, acc_sh, sem0, sem1):
    c = lax.axis_index("c")
    s = lax.axis_index("s")
    wid = s * NC + c
    _stage_rows(s, zero_hbm, acc_sh)
    plsc.subcore_barrier()
    _scatter_chunks(wid, (m0_hbm, m1_hbm, m2_hbm), (d0_hbm, d1_hbm, d2_hbm),
                    idx_v, rows0_v, rows1_v, acc_sh, sem0, sem1)
    plsc.subcore_barrier()
    _stage_rows(s, acc_sh, out_hbm, dst_off=c * N)


@functools.partial(pl.kernel, mesh=_mesh, out_type=_PART_T,
                   scratch_types=_SC_SCRATCH)
def _scatter2_sc(m3_hbm, m4_hbm, d3_hbm, d4_hbm, prev_hbm,
                 out_hbm, idx_v, rows0_v, rows1_v, acc_sh, sem0, sem1):
    c = lax.axis_index("c")
    s = lax.axis_index("s")
    wid = s * NC + c
    _stage_rows(s, prev_hbm, acc_sh, src_off=c * N)
    plsc.subcore_barrier()
    _scatter_chunks(wid, (m3_hbm, m4_hbm), (d3_hbm, d4_hbm),
                    idx_v, rows0_v, rows1_v, acc_sh, sem0, sem1)
    plsc.subcore_barrier()
    _stage_rows(s, acc_sh, out_hbm, dst_off=c * N)


_SQRT_HALF = 0.7071067811865476


def _msg_body(attr_ref, gath_ref, w_ref, wb_ref, bb_ref, out_ref):
    emb = lax.dot_general(attr_ref[...], wb_ref[...],
                          dimension_numbers=(((0,), (0,)), ((), ())),
                          preferred_element_type=jnp.float32) + bb_ref[...]
    u = gath_ref[...] + emb
    g = 0.5 * u * (1.0 + lax.erf(u * _SQRT_HALF))
    w = w_ref[pl.ds(pl.program_id(0) * _BE, _BE)]
    out_ref[...] = g * w[:, None]


def _fin_body(x_ref, p0_ref, p1_ref, epsb_ref, w1_ref, b1_ref, w2_ref,
              b2_ref, out_ref):
    h = x_ref[...] * epsb_ref[...] + p0_ref[0] + p1_ref[0]
    h1 = jnp.maximum(
        jnp.dot(h, w1_ref[...], preferred_element_type=jnp.float32)
        + b1_ref[...], 0.0)
    out_ref[...] = jnp.dot(h1, w2_ref[...],
                           preferred_element_type=jnp.float32) + b2_ref[...]


_BE = 6400
_BN = 2000


def _message_tc(gathered, edge_attr, edge_weight, Wb, bb2):
    grid = (ES // _BE,)
    return pl.pallas_call(
        _msg_body,
        grid=grid,
        in_specs=[
            pl.BlockSpec((DE, _BE), lambda i: (0, i)),
            pl.BlockSpec((_BE, D), lambda i: (i, 0)),
            pl.BlockSpec((ES,), lambda i: (0,)),
            pl.BlockSpec((DE, D), lambda i: (0, 0)),
            pl.BlockSpec((1, D), lambda i: (0, 0)),
        ],
        out_specs=pl.BlockSpec((_BE, D), lambda i: (i, 0)),
        out_shape=jax.ShapeDtypeStruct((ES, D), jnp.float32),
    )(edge_attr, gathered, edge_weight, Wb, bb2)


def _final_tc(x, parts, eps, W1, b1, W2, b2):
    grid = (N // _BN,)
    parts3 = parts.reshape(NC, N, D)
    epsb = jnp.broadcast_to((1.0 + eps).reshape(1, 1), (1, D))
    return pl.pallas_call(
        _fin_body,
        grid=grid,
        in_specs=[
            pl.BlockSpec((_BN, D), lambda i: (i, 0)),
            pl.BlockSpec((1, _BN, D), lambda i: (0, i, 0)),
            pl.BlockSpec((1, _BN, D), lambda i: (1, i, 0)),
            pl.BlockSpec((1, D), lambda i: (0, 0)),
            pl.BlockSpec((D, D), lambda i: (0, 0)),
            pl.BlockSpec((1, D), lambda i: (0, 0)),
            pl.BlockSpec((D, D), lambda i: (0, 0)),
            pl.BlockSpec((1, D), lambda i: (0, 0)),
        ],
        out_specs=pl.BlockSpec((_BN, D), lambda i: (i, 0)),
        out_shape=jax.ShapeDtypeStruct((N, D), jnp.float32),
    )(x, parts3, parts3, epsb, W1, b1.reshape(1, D), W2, b2.reshape(1, D))


def kernel(x, edge_index, edge_attr, edge_weight, eps, Wb, bb, W1, b1, W2, b2):
    src4 = edge_index[0].astype(jnp.int32).reshape(S, NW, RPW, K)
    dst4 = edge_index[1].astype(jnp.int32).reshape(S, NW, RPW, K)
    attrT = edge_attr.T
    w5 = edge_weight.reshape(S, ES)
    bb2 = bb.reshape(1, D)
    msgs = []
    for k in range(S):
        g_k = _gather_sc(x, src4[k])
        msgs.append(_message_tc(g_k, attrT[:, k * ES:(k + 1) * ES],
                                w5[k], Wb, bb2))
    zero = jnp.zeros((N, D), jnp.float32)
    parts_a = _scatter3_sc(msgs[0], msgs[1], msgs[2],
                           dst4[0], dst4[1], dst4[2], zero)
    parts = _scatter2_sc(msgs[3], msgs[4], dst4[3], dst4[4], parts_a)
    return _final_tc(x, parts, eps, W1, b1, W2, b2)
